# baseline (speedup 1.0000x reference)
import jax
import jax.numpy as jnp
from jax.experimental import pallas as pl

N = 4096
NFEAT = 512
NHID = 128
NCLASS = 64
HEADS = 4


def _proj_body(x_ref, w_ref, o_ref):
    o_ref[...] = jnp.dot(x_ref[...], w_ref[...],
                         preferred_element_type=jnp.float32)


def _proj(x, wcat):
    BM = 512
    return pl.pallas_call(
        _proj_body,
        grid=(N // BM,),
        in_specs=[
            pl.BlockSpec((BM, NFEAT), lambda i: (i, 0)),
            pl.BlockSpec((NFEAT, HEADS * NHID), lambda i: (0, 0)),
        ],
        out_specs=pl.BlockSpec((BM, HEADS * NHID), lambda i: (i, 0)),
        out_shape=jax.ShapeDtypeStruct((N, HEADS * NHID), jnp.float32),
    )(x, wcat)


def _maskmm_body(a_ref, b_ref, o_ref):
    acc = jnp.dot(a_ref[...], b_ref[...], preferred_element_type=jnp.float32)
    o_ref[...] = (acc > 0).astype(jnp.bfloat16)


def _maskmm(a, b):
    BM, BN = 512, 2048
    return pl.pallas_call(
        _maskmm_body,
        grid=(N // BM, N // BN),
        in_specs=[
            pl.BlockSpec((BM, N), lambda i, j: (i, 0)),
            pl.BlockSpec((N, BN), lambda i, j: (0, j)),
        ],
        out_specs=pl.BlockSpec((BM, BN), lambda i, j: (i, j)),
        out_shape=jax.ShapeDtypeStruct((N, N), jnp.bfloat16),
    )(a, b)


def _gat_body(h_ref, hf_ref, a_ref, m_ref, o_ref):
    h = h_ref[...]
    hfull = hf_ref[...]
    a1 = a_ref[0:1, :]
    a2 = a_ref[1:2, :]
    f1 = jnp.sum(h * a1, axis=1, keepdims=True)
    f2 = jnp.sum(hfull * a2, axis=1, keepdims=True)
    e = f1 + f2.T
    e = jnp.where(e > 0, e, 0.2 * e)
    e = jnp.where(m_ref[...] > 0, e, -9e15)
    mx = jnp.max(e, axis=1, keepdims=True)
    p = jnp.exp(e - mx)
    s = jnp.sum(p, axis=1, keepdims=True)
    out = jnp.dot(p, hfull, preferred_element_type=jnp.float32)
    o_ref[...] = out / s


def _gat(h, a2d, mask):
    BM = 512
    return pl.pallas_call(
        _gat_body,
        grid=(N // BM,),
        in_specs=[
            pl.BlockSpec((BM, NHID), lambda i: (i, 0)),
            pl.BlockSpec((N, NHID), lambda i: (0, 0)),
            pl.BlockSpec((2, NHID), lambda i: (0, 0)),
            pl.BlockSpec((BM, N), lambda i: (i, 0)),
        ],
        out_specs=pl.BlockSpec((BM, NHID), lambda i: (i, 0)),
        out_shape=jax.ShapeDtypeStruct((N, NHID), jnp.float32),
    )(h, h, a2d, mask)


def _final_body(h_ref, w_ref, b_ref, o_ref):
    h = jnp.maximum(h_ref[...], 0.0)
    logits = jnp.dot(h, w_ref[...], preferred_element_type=jnp.float32)
    logits = logits + b_ref[...]
    mx = jnp.max(logits, axis=1, keepdims=True)
    l = logits - mx
    lse = jnp.log(jnp.sum(jnp.exp(l), axis=1, keepdims=True))
    o_ref[...] = l - lse


def _final(hcat, fc_wt, fc_b2d):
    BM = 512
    return pl.pallas_call(
        _final_body,
        grid=(N // BM,),
        in_specs=[
            pl.BlockSpec((BM, HEADS * NHID), lambda i: (i, 0)),
            pl.BlockSpec((HEADS * NHID, NCLASS), lambda i: (0, 0)),
            pl.BlockSpec((1, NCLASS), lambda i: (0, 0)),
        ],
        out_specs=pl.BlockSpec((BM, NCLASS), lambda i: (i, 0)),
        out_shape=jax.ShapeDtypeStruct((N, NCLASS), jnp.float32),
    )(hcat, fc_wt, fc_b2d)


def kernel(x, adj, W, a, fc_w, fc_b):
    m1 = (adj > 0).astype(jnp.bfloat16)
    m2 = _maskmm(m1, m1)
    m3 = _maskmm(m2, m1)

    wcat = jnp.concatenate([W[HEADS - 1], W[0], W[1], W[2]], axis=1)
    H = _proj(x, wcat)
    h3 = H[:, 0:NHID]
    hs = [H[:, NHID * (i + 1):NHID * (i + 2)] for i in range(HEADS - 1)]

    masks = [m1, m2, m3]
    outs = [h3]
    for i in range(HEADS - 1):
        a2d = a[i].reshape(2, NHID)
        outs.append(_gat(hs[i], a2d, masks[i]))

    hcat = jnp.concatenate(outs, axis=1)
    return _final(hcat, fc_w.T, fc_b.reshape(1, NCLASS))
